# Initial kernel scaffold; baseline (speedup 1.0000x reference)
import jax
import jax.numpy as jnp
from jax.experimental import pallas as pl


def kernel(weight_origin, ci_1, ci_2, ci_3, ci_4, ci_5, review_feat_1, review_feat_2, review_feat_3, review_feat_4, review_feat_5, Wr_1, Wr_2, Wr_3, Wr_4, Wr_5, fc_user_w, fc_user_b, fc_item_w, fc_item_b, pred_w1, pred_w2, edge_index_1, edge_index_2, edge_index_3, edge_index_4, edge_index_5, users, items):
    raise NotImplementedError("write your pallas kernel here")


# speedup vs baseline: 9.5386x; 9.5386x over previous
import functools

import jax
import jax.numpy as jnp
from jax import lax
from jax.experimental import pallas as pl
from jax.experimental.pallas import tpu as pltpu
from jax.experimental.pallas import tpu_sc as plsc

NUM_USERS = 3000
NUM_ITEMS = 7000
N = NUM_USERS + NUM_ITEMS
E = 320000
EMB = 16
REV = 128
D = (EMB + REV) * 5
B = 4096

NC = 2
NS = 16
NW = NC * NS
EPT = E // NW
CH = 80
NCHUNK = EPT // CH
ZBLK = 400
NBLK = N // ZBLK

_HI = jax.lax.Precision.HIGHEST


def _mesh():
    return plsc.VectorSubcoreMesh(
        core_axis_name="c", subcore_axis_name="s", num_cores=NC,
        num_subcores=NS)


def _g_body(wo_ref, ci5_ref, g_ref):
    wo = wo_ref[...]
    ci5 = ci5_ref[...]
    cols = [wo[:, 16 * r:16 * (r + 1)] * ci5[:, r:r + 1] for r in range(5)]
    g_ref[...] = jnp.concatenate(cols, axis=1)


def _stage0(weight_origin, ci5):
    blk = 1000
    return pl.pallas_call(
        _g_body,
        grid=(N // blk,),
        in_specs=[
            pl.BlockSpec((blk, 80), lambda i: (i, 0)),
            pl.BlockSpec((blk, 5), lambda i: (i, 0)),
        ],
        out_specs=pl.BlockSpec((blk, 80), lambda i: (i, 0)),
        out_shape=jax.ShapeDtypeStruct((N, 80), jnp.float32),
    )(weight_origin, ci5)


def _scatter_body(src3d, dst3d, rev, ci, g, out64, out16,
                  srcb, dstb, revb, gb, civ, z64, z16,
                  acc64, acc16, sem, semc):
    cid = lax.axis_index("c")
    sid = lax.axis_index("s")

    zv = jnp.zeros((16,), jnp.float32)

    def zrow(i, carry):
        for k in range(4):
            z64[i, pl.ds(16 * k, 16)] = zv
        z16[i, :] = zv
        return carry

    lax.fori_loop(0, ZBLK, zrow, 0)
    for rep in range(2):
        bid = sid + NS * rep
        @pl.when(bid < NBLK)
        def _():
            off = pl.multiple_of(bid * ZBLK, 8)
            pltpu.sync_copy(z64, acc64.at[pl.ds(off, ZBLK)])
            pltpu.sync_copy(z16, acc16.at[pl.ds(off, ZBLK)])

    cw = cid * NS + sid
    pltpu.sync_copy(src3d.at[cw], srcb)
    pltpu.sync_copy(dst3d.at[cw], dstb)
    plsc.subcore_barrier()

    base_e = cw * EPT

    def chunk(c, carry):
        e0 = pl.multiple_of(base_e + c * CH, 8)
        pltpu.sync_copy(rev.at[pl.ds(e0, CH)], revb)
        ccopy = pltpu.async_copy(ci.at[srcb.at[c]], civ, semc)
        gcopy = pltpu.async_copy(g.at[srcb.at[c]], gb, sem)
        ccopy.wait()
        for grp in range(CH // 16):
            ci16 = civ[pl.ds(16 * grp, 16)]
            for e in range(16):
                row = 16 * grp + e
                ce = ci16[e]
                for k in range(4):
                    revb[row, pl.ds(16 * k, 16)] = (
                        revb[row, pl.ds(16 * k, 16)] * ce)
        gcopy.wait()
        pltpu.sync_copy(revb, acc64.at[dstb.at[c]], add=True)
        pltpu.sync_copy(gb, acc16.at[dstb.at[c]], add=True)
        return carry

    lax.fori_loop(0, NCHUNK, chunk, 0)
    plsc.subcore_barrier()

    for rep in range(2):
        bid = sid + NS * rep
        @pl.when(bid < NBLK)
        def _():
            off = pl.multiple_of(bid * ZBLK, 8)
            pltpu.sync_copy(acc64.at[pl.ds(off, ZBLK)],
                            out64.at[cid, pl.ds(off, ZBLK)])
            pltpu.sync_copy(acc16.at[pl.ds(off, ZBLK)],
                            out16.at[cid, pl.ds(off, ZBLK)])


def _stage1(src3d, dst3d, rev, ci, g):
    return pl.kernel(
        _scatter_body,
        out_type=[
            jax.ShapeDtypeStruct((NC, N, 64), jnp.float32),
            jax.ShapeDtypeStruct((NC, N, 16), jnp.float32),
        ],
        mesh=_mesh(),
        compiler_params=pltpu.CompilerParams(use_tc_tiling_on_sc=False),
        scratch_types=[
            pltpu.VMEM((NCHUNK, CH), jnp.int32),
            pltpu.VMEM((NCHUNK, CH), jnp.int32),
            pltpu.VMEM((CH, 64), jnp.float32),
            pltpu.VMEM((CH, 16), jnp.float32),
            pltpu.VMEM((CH,), jnp.float32),
            pltpu.VMEM((ZBLK, 64), jnp.float32),
            pltpu.VMEM((ZBLK, 16), jnp.float32),
            pltpu.VMEM_SHARED((N, 64), jnp.float32),
            pltpu.VMEM_SHARED((N, 16), jnp.float32),
            pltpu.SemaphoreType.DMA,
            pltpu.SemaphoreType.DMA,
        ],
    )(src3d, dst3d, rev, ci, g)


def _dense_body(o64_1, o64_2, o64_3, o64_4, o64_5,
                o16_1, o16_2, o16_3, o16_4, o16_5,
                wr_ref, ci5_ref, fcw_ref, fcb_ref, out_ref):
    o64s = [o64_1, o64_2, o64_3, o64_4, o64_5]
    o16s = [o16_1, o16_2, o16_3, o16_4, o16_5]
    ci5 = ci5_ref[...]
    cols = []
    for r in range(5):
        a64 = o64s[r][0] + o64s[r][1]
        a16 = o16s[r][0] + o16s[r][1]
        h = lax.dot_general(a64, wr_ref[r], (((1,), (1,)), ((), ())),
                            precision=_HI,
                            preferred_element_type=jnp.float32)
        cr = ci5[:, r:r + 1]
        cols.append(h * cr)
        cols.append(a16 * cr)
    feat = jnp.concatenate(cols, axis=1)
    w = fcw_ref[0]
    feat2 = lax.dot_general(feat, w, (((1,), (1,)), ((), ())),
                            precision=_HI,
                            preferred_element_type=jnp.float32)
    out_ref[...] = feat2 + fcb_ref[0, 0]


def _stage2(o64s, o16s, wr_all, ci5, fcw_all, fcb_all):
    blk = 1000
    nub = NUM_USERS // blk
    sel = lambda i: jnp.where(i < nub, 0, 1)
    return pl.pallas_call(
        _dense_body,
        grid=(N // blk,),
        in_specs=(
            [pl.BlockSpec((NC, blk, 64), lambda i: (0, i, 0))] * 5
            + [pl.BlockSpec((NC, blk, 16), lambda i: (0, i, 0))] * 5
            + [
                pl.BlockSpec((5, 128, 64), lambda i: (0, 0, 0)),
                pl.BlockSpec((blk, 5), lambda i: (i, 0)),
                pl.BlockSpec((1, D, D), lambda i: (sel(i), 0, 0)),
                pl.BlockSpec((1, 1, D), lambda i: (sel(i), 0, 0)),
            ]
        ),
        out_specs=pl.BlockSpec((blk, D), lambda i: (i, 0)),
        out_shape=jax.ShapeDtypeStruct((N, D), jnp.float32),
    )(*o64s, *o16s, wr_all, ci5, fcw_all, fcb_all)


_B_ROWS = B // 16
_RPW = _B_ROWS // NW


def _pair_body(feat2, users2d, items2d, q, uixb, iixb, ub, ib, qb,
               semu, semi):
    cid = lax.axis_index("c")
    sid = lax.axis_index("s")
    wid = cid * NS + sid
    w0 = pl.multiple_of(wid * _RPW, 8)
    pltpu.sync_copy(users2d.at[pl.ds(w0, _RPW)], uixb)
    pltpu.sync_copy(items2d.at[pl.ds(w0, _RPW)], iixb)

    def chunk(c, carry):
        cu = pltpu.async_copy(feat2.at[uixb.at[c]], ub, semu)
        cv = pltpu.async_copy(feat2.at[iixb.at[c]], ib, semi)
        cu.wait()
        cv.wait()

        def rowf(rw, carry2):
            for k in range(D // 16):
                qb[rw, pl.ds(16 * k, 16)] = (
                    ub[rw, pl.ds(16 * k, 16)] * ib[rw, pl.ds(16 * k, 16)])
            return carry2

        lax.fori_loop(0, 16, rowf, 0)
        q0 = pl.multiple_of(wid * 16 * _RPW + c * 16, 8)
        pltpu.sync_copy(qb, q.at[pl.ds(q0, 16)])
        return carry

    lax.fori_loop(0, _RPW, chunk, 0)


def _stage3(feat2, users2d, items2d):
    return pl.kernel(
        _pair_body,
        out_type=jax.ShapeDtypeStruct((B, D), jnp.float32),
        mesh=_mesh(),
        compiler_params=pltpu.CompilerParams(use_tc_tiling_on_sc=False),
        scratch_types=[
            pltpu.VMEM((_RPW, 16), jnp.int32),
            pltpu.VMEM((_RPW, 16), jnp.int32),
            pltpu.VMEM((16, D), jnp.float32),
            pltpu.VMEM((16, D), jnp.float32),
            pltpu.VMEM((16, D), jnp.float32),
            pltpu.SemaphoreType.DMA,
            pltpu.SemaphoreType.DMA,
        ],
    )(feat2, users2d, items2d)


def _pred_body(q_ref, w1_ref, w2_ref, out_ref):
    h = lax.dot_general(q_ref[...], w1_ref[...], (((1,), (1,)), ((), ())),
                        precision=_HI, preferred_element_type=jnp.float32)
    h = jnp.maximum(h, 0.0)
    out_ref[...] = lax.dot_general(h, w2_ref[...], (((1,), (1,)), ((), ())),
                                   precision=_HI,
                                   preferred_element_type=jnp.float32)


def _stage4(q, w1, w2):
    blk = 512
    return pl.pallas_call(
        _pred_body,
        grid=(B // blk,),
        in_specs=[
            pl.BlockSpec((blk, D), lambda i: (i, 0)),
            pl.BlockSpec((D, D), lambda i: (0, 0)),
            pl.BlockSpec((5, D), lambda i: (0, 0)),
        ],
        out_specs=pl.BlockSpec((blk, 5), lambda i: (i, 0)),
        out_shape=jax.ShapeDtypeStruct((B, 5), jnp.float32),
    )(q, w1, w2)


def kernel(weight_origin, ci_1, ci_2, ci_3, ci_4, ci_5,
           review_feat_1, review_feat_2, review_feat_3, review_feat_4,
           review_feat_5, Wr_1, Wr_2, Wr_3, Wr_4, Wr_5,
           fc_user_w, fc_user_b, fc_item_w, fc_item_b, pred_w1, pred_w2,
           edge_index_1, edge_index_2, edge_index_3, edge_index_4,
           edge_index_5, users, items):
    cis = [ci_1, ci_2, ci_3, ci_4, ci_5]
    rfs = [review_feat_1, review_feat_2, review_feat_3, review_feat_4,
           review_feat_5]
    eis = [edge_index_1, edge_index_2, edge_index_3, edge_index_4,
           edge_index_5]
    ci5 = jnp.concatenate(cis, axis=1)
    g_all = _stage0(weight_origin, ci5)

    o64s, o16s = [], []
    for r in range(5):
        src3d = eis[r][0].reshape(NW, NCHUNK, CH)
        dst3d = eis[r][1].reshape(NW, NCHUNK, CH)
        o64, o16 = _stage1(src3d, dst3d, rfs[r], cis[r][:, 0],
                           g_all[:, 16 * r:16 * (r + 1)])
        o64s.append(o64)
        o16s.append(o16)

    wr_all = jnp.stack([Wr_1, Wr_2, Wr_3, Wr_4, Wr_5])
    fcw_all = jnp.stack([fc_user_w, fc_item_w])
    fcb_all = jnp.stack([fc_user_b, fc_item_b])[:, None, :]
    feat2 = _stage2(o64s, o16s, wr_all, ci5, fcw_all, fcb_all)

    q = _stage3(feat2, users.reshape(_B_ROWS, 16),
                items.reshape(_B_ROWS, 16))
    out = _stage4(q, pred_w1, pred_w2)
    return out
